# Initial kernel scaffold; baseline (speedup 1.0000x reference)
import functools

import jax
import jax.numpy as jnp
from jax import lax
from jax.experimental import pallas as pl
from jax.experimental.pallas import tpu as pltpu
from jax.experimental.pallas import tpu_sc as plsc

N_NODES = 10000
N_EDGES = 320000
N_GRAPHS = 64

CH = 128
CPT = 80
N_TILES = 32
N_CHUNKS = N_TILES * CPT
E_PAD = N_CHUNKS * CH
DUMMY_ROW = N_NODES
ACC_ROWS = 10016
RPT = ACC_ROWS // 16


def _make_agg(d):
    mesh = plsc.VectorSubcoreMesh(core_axis_name="c", subcore_axis_name="s")

    @functools.partial(
        pl.kernel,
        mesh=mesh,
        out_type=jax.ShapeDtypeStruct((2, ACC_ROWS, d), jnp.float32),
        scratch_types=[
            pltpu.VMEM((CPT, CH), jnp.int32),
            pltpu.VMEM((CPT, CH), jnp.int32),
            pltpu.VMEM((CH, d), jnp.float32),
            pltpu.VMEM_SHARED((ACC_ROWS, d), jnp.float32),
            pltpu.SemaphoreType.DMA,
        ],
    )
    def agg(p_hbm, src_hbm, dst_hbm, z_hbm, out_hbm, src_v, dst_v, buf, acc, sem):
        c = lax.axis_index("c")
        s = lax.axis_index("s")
        wid = c * 16 + s
        pltpu.sync_copy(z_hbm.at[pl.ds(s * RPT, RPT)], acc.at[pl.ds(s * RPT, RPT)])
        pltpu.sync_copy(src_hbm.at[pl.ds(wid * CPT, CPT)], src_v)
        pltpu.sync_copy(dst_hbm.at[pl.ds(wid * CPT, CPT)], dst_v)
        plsc.subcore_barrier()

        def body(j, carry):
            pltpu.async_copy(p_hbm.at[src_v.at[j]], buf, sem).wait()
            pltpu.sync_copy(buf, acc.at[dst_v.at[j]], add=True)
            return carry

        lax.fori_loop(0, CPT, body, 0)
        plsc.subcore_barrier()
        pltpu.sync_copy(acc.at[pl.ds(s * RPT, RPT)],
                        out_hbm.at[c, pl.ds(s * RPT, RPT)])

    return agg


_AGG_CACHE = {}


def _get_agg(d):
    if d not in _AGG_CACHE:
        _AGG_CACHE[d] = _make_agg(d)
    return _AGG_CACHE[d]


def _dot_t(a, w):
    return lax.dot_general(a, w, (((1,), (1,)), ((), ())),
                           preferred_element_type=jnp.float32)


def _proj1_body(x_ref, w_rel_ref, w_root_ref, p_ref, r_ref):
    x = x_ref[...]
    p_ref[...] = _dot_t(x, w_rel_ref[...])
    r_ref[...] = _dot_t(x, w_root_ref[...])


def _mid1_body(parts_ref, r1_ref, b1_ref, g1_ref, be1_ref,
               w2rel_ref, w2root_ref, p2_ref, r2_ref):
    agg = parts_ref[0, :N_NODES, :] + parts_ref[1, :N_NODES, :]
    h = agg + b1_ref[...] + r1_ref[...]
    mean = jnp.mean(h, axis=0, keepdims=True)
    var = jnp.mean((h - mean) ** 2, axis=0, keepdims=True)
    h = (h - mean) / jnp.sqrt(var + 1e-5) * g1_ref[...] + be1_ref[...]
    h = jnp.maximum(h, 0.0)
    p2_ref[...] = _dot_t(h, w2rel_ref[...])
    r2_ref[...] = _dot_t(h, w2root_ref[...])


def _mid2_body(parts_ref, r2_ref, b2_ref, w3rel_ref, w3root_ref,
               p3_ref, r3_ref):
    agg = parts_ref[0, :N_NODES, :] + parts_ref[1, :N_NODES, :]
    h = jnp.maximum(agg + b2_ref[...] + r2_ref[...], 0.0)
    p3_ref[...] = _dot_t(h, w3rel_ref[...])
    r3_ref[...] = _dot_t(h, w3root_ref[...])


def _final_body(parts_ref, r3_ref, b3_ref, g2_ref, be2_ref, batch_ref,
                linw_ref, linb_ref, out_ref):
    agg = parts_ref[0, :N_NODES, :] + parts_ref[1, :N_NODES, :]
    h = agg + b3_ref[...] + r3_ref[...]
    mean = jnp.mean(h, axis=0, keepdims=True)
    var = jnp.mean((h - mean) ** 2, axis=0, keepdims=True)
    h = (h - mean) / jnp.sqrt(var + 1e-5) * g2_ref[...] + be2_ref[...]
    onehot = (batch_ref[...] ==
              lax.broadcasted_iota(jnp.int32, (N_NODES, N_GRAPHS), 1)
              ).astype(jnp.float32)
    sums = lax.dot_general(onehot, h, (((0,), (0,)), ((), ())),
                           preferred_element_type=jnp.float32)
    counts = jnp.sum(onehot, axis=0)
    means = sums / jnp.maximum(counts, 1.0)[:, None]
    out_ref[...] = _dot_t(means, linw_ref[...]) + linb_ref[...]


def _tc(body, out_shapes, *args):
    return pl.pallas_call(body, out_shape=out_shapes)(*args)


def kernel(x, edge_index, batch, W1_rel, b1_rel, W1_root, bn1_gamma, bn1_beta,
           W2_rel, b2_rel, W2_root, W3_rel, b3_rel, W3_root,
           bn2_gamma, bn2_beta, lin_W, lin_b):
    f32 = jnp.float32
    src = edge_index[0].astype(jnp.int32)
    dst = edge_index[1].astype(jnp.int32)
    pad = E_PAD - N_EDGES
    src2d = jnp.concatenate([src, jnp.zeros((pad,), jnp.int32)]).reshape(N_CHUNKS, CH)
    dst2d = jnp.concatenate([dst, jnp.full((pad,), DUMMY_ROW, jnp.int32)]).reshape(N_CHUNKS, CH)
    z64 = jnp.zeros((ACC_ROWS, 64), f32)
    z32 = jnp.zeros((ACC_ROWS, 32), f32)
    batch2 = batch.astype(jnp.int32).reshape(N_NODES, 1)

    w3rel = jnp.pad(W3_rel, ((0, 12), (0, 0)))
    w3root = jnp.pad(W3_root, ((0, 12), (0, 0)))
    b3 = jnp.pad(b3_rel, (0, 12)).reshape(1, 32)
    g2 = jnp.pad(bn2_gamma, (0, 12)).reshape(1, 32)
    be2 = jnp.pad(bn2_beta, (0, 12)).reshape(1, 32)
    linw = jnp.pad(lin_W, ((0, 0), (0, 12)))

    sds = jax.ShapeDtypeStruct

    p1, r1 = _tc(_proj1_body,
                 [sds((N_NODES, 64), f32), sds((N_NODES, 64), f32)],
                 x, W1_rel, W1_root)
    parts1 = _get_agg(64)(p1, src2d, dst2d, z64)
    p2, r2 = _tc(_mid1_body,
                 [sds((N_NODES, 32), f32), sds((N_NODES, 32), f32)],
                 parts1, r1, b1_rel.reshape(1, 64),
                 bn1_gamma.reshape(1, 64), bn1_beta.reshape(1, 64),
                 W2_rel, W2_root)
    parts2 = _get_agg(32)(p2, src2d, dst2d, z32)
    p3, r3 = _tc(_mid2_body,
                 [sds((N_NODES, 32), f32), sds((N_NODES, 32), f32)],
                 parts2, r2, b2_rel.reshape(1, 32), w3rel, w3root)
    parts3 = _get_agg(32)(p3, src2d, dst2d, z32)
    out = _tc(_final_body, sds((N_GRAPHS, 11), f32),
              parts3, r3, b3, g2, be2, batch2,
              linw, lin_b.reshape(1, 11))
    return out


# speedup vs baseline: 6.2611x; 6.2611x over previous
import functools

import jax
import jax.numpy as jnp
from jax import lax
from jax.experimental import pallas as pl
from jax.experimental.pallas import tpu as pltpu
from jax.experimental.pallas import tpu_sc as plsc

N_NODES = 10000
N_EDGES = 320000
N_GRAPHS = 64

CH = 128
CPT = 80
N_TILES = 32
N_CHUNKS = N_TILES * CPT
E_PAD = N_CHUNKS * CH
DUMMY_ROW = N_NODES
ACC_ROWS = 10112
RPT = ACC_ROWS // 16


def _make_agg(d):
    mesh = plsc.VectorSubcoreMesh(core_axis_name="c", subcore_axis_name="s")

    @functools.partial(
        pl.kernel,
        mesh=mesh,
        compiler_params=pltpu.CompilerParams(use_tc_tiling_on_sc=False),
        out_type=jax.ShapeDtypeStruct((2, ACC_ROWS, d), jnp.float32),
        scratch_types=[
            pltpu.VMEM((CPT, CH), jnp.int32),
            pltpu.VMEM((CPT, CH), jnp.int32),
            pltpu.VMEM((CH, d), jnp.float32),
            pltpu.VMEM_SHARED((ACC_ROWS, d), jnp.float32),
            pltpu.SemaphoreType.DMA,
        ],
    )
    def agg(p_hbm, src_hbm, dst_hbm, z_hbm, out_hbm, src_v, dst_v, buf, acc, sem):
        c = lax.axis_index("c")
        s = lax.axis_index("s")
        wid = c * 16 + s
        pltpu.sync_copy(z_hbm.at[pl.ds(s * RPT, RPT)], acc.at[pl.ds(s * RPT, RPT)])
        pltpu.sync_copy(src_hbm.at[pl.ds(wid * CPT, CPT)], src_v)
        pltpu.sync_copy(dst_hbm.at[pl.ds(wid * CPT, CPT)], dst_v)
        plsc.subcore_barrier()

        def body(j, carry):
            pltpu.async_copy(p_hbm.at[src_v.at[j]], buf, sem).wait()
            pltpu.sync_copy(buf, acc.at[dst_v.at[j]], add=True)
            return carry

        lax.fori_loop(0, CPT, body, 0)
        plsc.subcore_barrier()
        pltpu.sync_copy(acc.at[pl.ds(s * RPT, RPT)],
                        out_hbm.at[c, pl.ds(s * RPT, RPT)])

    return agg


_AGG_CACHE = {}


def _get_agg(d):
    if d not in _AGG_CACHE:
        _AGG_CACHE[d] = _make_agg(d)
    return _AGG_CACHE[d]


def _dot_t(a, w):
    return lax.dot_general(a, w, (((1,), (1,)), ((), ())),
                           preferred_element_type=jnp.float32)


def _proj1_body(x_ref, w_rel_ref, w_root_ref, p_ref, r_ref):
    x = x_ref[...]
    p_ref[...] = _dot_t(x, w_rel_ref[...])
    r_ref[...] = _dot_t(x, w_root_ref[...])


def _mid1_body(parts_ref, r1_ref, b1_ref, g1_ref, be1_ref,
               w2rel_ref, w2root_ref, p2_ref, r2_ref):
    agg = parts_ref[0, :N_NODES, :] + parts_ref[1, :N_NODES, :]
    h = agg + b1_ref[...] + r1_ref[...]
    mean = jnp.mean(h, axis=0, keepdims=True)
    var = jnp.mean((h - mean) ** 2, axis=0, keepdims=True)
    h = (h - mean) / jnp.sqrt(var + 1e-5) * g1_ref[...] + be1_ref[...]
    h = jnp.maximum(h, 0.0)
    p2_ref[...] = _dot_t(h, w2rel_ref[...])
    r2_ref[...] = _dot_t(h, w2root_ref[...])


def _mid2_body(parts_ref, r2_ref, b2_ref, w3rel_ref, w3root_ref,
               p3_ref, r3_ref):
    agg = parts_ref[0, :N_NODES, :] + parts_ref[1, :N_NODES, :]
    h = jnp.maximum(agg + b2_ref[...] + r2_ref[...], 0.0)
    p3_ref[...] = _dot_t(h, w3rel_ref[...])
    r3_ref[...] = _dot_t(h, w3root_ref[...])


def _final_body(parts_ref, r3_ref, b3_ref, g2_ref, be2_ref, batch_ref,
                linw_ref, linb_ref, out_ref):
    agg = parts_ref[0, :N_NODES, :] + parts_ref[1, :N_NODES, :]
    h = agg + b3_ref[...] + r3_ref[...]
    mean = jnp.mean(h, axis=0, keepdims=True)
    var = jnp.mean((h - mean) ** 2, axis=0, keepdims=True)
    h = (h - mean) / jnp.sqrt(var + 1e-5) * g2_ref[...] + be2_ref[...]
    onehot = (batch_ref[...] ==
              lax.broadcasted_iota(jnp.int32, (N_NODES, N_GRAPHS), 1)
              ).astype(jnp.float32)
    sums = lax.dot_general(onehot, h, (((0,), (0,)), ((), ())),
                           preferred_element_type=jnp.float32)
    counts = jnp.sum(onehot, axis=0)
    means = sums / jnp.maximum(counts, 1.0)[:, None]
    out_ref[...] = _dot_t(means, linw_ref[...]) + linb_ref[...]


def _tc(body, out_shapes, *args):
    return pl.pallas_call(body, out_shape=out_shapes)(*args)


def kernel(x, edge_index, batch, W1_rel, b1_rel, W1_root, bn1_gamma, bn1_beta,
           W2_rel, b2_rel, W2_root, W3_rel, b3_rel, W3_root,
           bn2_gamma, bn2_beta, lin_W, lin_b):
    f32 = jnp.float32
    src = edge_index[0].astype(jnp.int32)
    dst = edge_index[1].astype(jnp.int32)
    pad = E_PAD - N_EDGES
    src2d = jnp.concatenate([src, jnp.zeros((pad,), jnp.int32)]).reshape(N_CHUNKS, CH)
    dst2d = jnp.concatenate([dst, jnp.full((pad,), DUMMY_ROW, jnp.int32)]).reshape(N_CHUNKS, CH)
    z64 = jnp.zeros((ACC_ROWS, 64), f32)
    z32 = jnp.zeros((ACC_ROWS, 32), f32)
    batch2 = batch.astype(jnp.int32).reshape(N_NODES, 1)

    w3rel = jnp.pad(W3_rel, ((0, 12), (0, 0)))
    w3root = jnp.pad(W3_root, ((0, 12), (0, 0)))
    b3 = jnp.pad(b3_rel, (0, 12)).reshape(1, 32)
    g2 = jnp.pad(bn2_gamma, (0, 12)).reshape(1, 32)
    be2 = jnp.pad(bn2_beta, (0, 12)).reshape(1, 32)
    linw = jnp.pad(lin_W, ((0, 0), (0, 12)))

    sds = jax.ShapeDtypeStruct

    p1, r1 = _tc(_proj1_body,
                 [sds((N_NODES, 64), f32), sds((N_NODES, 64), f32)],
                 x, W1_rel, W1_root)
    parts1 = _get_agg(64)(p1, src2d, dst2d, z64)
    p2, r2 = _tc(_mid1_body,
                 [sds((N_NODES, 32), f32), sds((N_NODES, 32), f32)],
                 parts1, r1, b1_rel.reshape(1, 64),
                 bn1_gamma.reshape(1, 64), bn1_beta.reshape(1, 64),
                 W2_rel, W2_root)
    parts2 = _get_agg(32)(p2, src2d, dst2d, z32)
    p3, r3 = _tc(_mid2_body,
                 [sds((N_NODES, 32), f32), sds((N_NODES, 32), f32)],
                 parts2, r2, b2_rel.reshape(1, 32), w3rel, w3root)
    parts3 = _get_agg(32)(p3, src2d, dst2d, z32)
    out = _tc(_final_body, sds((N_GRAPHS, 11), f32),
              parts3, r3, b3, g2, be2, batch2,
              linw, lin_b.reshape(1, 11))
    return out


# speedup vs baseline: 6.9632x; 1.1121x over previous
import functools

import jax
import jax.numpy as jnp
from jax import lax
from jax.experimental import pallas as pl
from jax.experimental.pallas import tpu as pltpu
from jax.experimental.pallas import tpu_sc as plsc

N_NODES = 10000
N_EDGES = 320000
N_GRAPHS = 64

CH = 128
CPT = 80
N_TILES = 32
N_CHUNKS = N_TILES * CPT
E_PAD = N_CHUNKS * CH
DUMMY_ROW = N_NODES
ACC_ROWS = 10112
RPT = ACC_ROWS // 16


def _make_agg(d):
    mesh = plsc.VectorSubcoreMesh(core_axis_name="c", subcore_axis_name="s")

    @functools.partial(
        pl.kernel,
        mesh=mesh,
        compiler_params=pltpu.CompilerParams(use_tc_tiling_on_sc=False),
        out_type=jax.ShapeDtypeStruct((2, ACC_ROWS, d), jnp.float32),
        scratch_types=[
            pltpu.VMEM((CPT, CH), jnp.int32),
            pltpu.VMEM((CPT, CH), jnp.int32),
            [pltpu.VMEM((CH, d), jnp.float32) for _ in range(4)],
            pltpu.VMEM_SHARED((ACC_ROWS, d), jnp.float32),
            [pltpu.SemaphoreType.DMA for _ in range(4)],
        ],
    )
    def agg(p_hbm, src_hbm, dst_hbm, z_hbm, out_hbm,
            src_v, dst_v, bufs, acc, sems):
        c = lax.axis_index("c")
        s = lax.axis_index("s")
        wid = c * 16 + s
        pltpu.sync_copy(z_hbm.at[pl.ds(s * RPT, RPT)], acc.at[pl.ds(s * RPT, RPT)])
        pltpu.sync_copy(src_hbm.at[pl.ds(wid * CPT, CPT)], src_v)
        pltpu.sync_copy(dst_hbm.at[pl.ds(wid * CPT, CPT)], dst_v)
        plsc.subcore_barrier()

        def body(jj, carry):
            j = jj * 4
            hs = [pltpu.async_copy(p_hbm.at[src_v.at[j + b]], bufs[b], sems[b])
                  for b in range(4)]
            for b in range(4):
                hs[b].wait()
                pltpu.sync_copy(bufs[b], acc.at[dst_v.at[j + b]], add=True)
            return carry

        lax.fori_loop(0, CPT // 4, body, 0)
        plsc.subcore_barrier()
        pltpu.sync_copy(acc.at[pl.ds(s * RPT, RPT)],
                        out_hbm.at[c, pl.ds(s * RPT, RPT)])

    return agg


_AGG_CACHE = {}


def _get_agg(d):
    if d not in _AGG_CACHE:
        _AGG_CACHE[d] = _make_agg(d)
    return _AGG_CACHE[d]


def _dot_t(a, w):
    return lax.dot_general(a, w, (((1,), (1,)), ((), ())),
                           preferred_element_type=jnp.float32)


def _proj1_body(x_ref, w_rel_ref, w_root_ref, p_ref, r_ref):
    x = x_ref[...]
    p_ref[...] = _dot_t(x, w_rel_ref[...])
    r_ref[...] = _dot_t(x, w_root_ref[...])


def _mid1_body(parts_ref, r1_ref, b1_ref, g1_ref, be1_ref,
               w2rel_ref, w2root_ref, p2_ref, r2_ref):
    agg = parts_ref[0, :N_NODES, :] + parts_ref[1, :N_NODES, :]
    h = agg + b1_ref[...] + r1_ref[...]
    mean = jnp.mean(h, axis=0, keepdims=True)
    var = jnp.mean((h - mean) ** 2, axis=0, keepdims=True)
    h = (h - mean) / jnp.sqrt(var + 1e-5) * g1_ref[...] + be1_ref[...]
    h = jnp.maximum(h, 0.0)
    p2_ref[...] = _dot_t(h, w2rel_ref[...])
    r2_ref[...] = _dot_t(h, w2root_ref[...])


def _mid2_body(parts_ref, r2_ref, b2_ref, w3rel_ref, w3root_ref,
               p3_ref, r3_ref):
    agg = parts_ref[0, :N_NODES, :] + parts_ref[1, :N_NODES, :]
    h = jnp.maximum(agg + b2_ref[...] + r2_ref[...], 0.0)
    p3_ref[...] = _dot_t(h, w3rel_ref[...])
    r3_ref[...] = _dot_t(h, w3root_ref[...])


def _final_body(parts_ref, r3_ref, b3_ref, g2_ref, be2_ref, batch_ref,
                linw_ref, linb_ref, out_ref):
    agg = parts_ref[0, :N_NODES, :] + parts_ref[1, :N_NODES, :]
    h = agg + b3_ref[...] + r3_ref[...]
    mean = jnp.mean(h, axis=0, keepdims=True)
    var = jnp.mean((h - mean) ** 2, axis=0, keepdims=True)
    h = (h - mean) / jnp.sqrt(var + 1e-5) * g2_ref[...] + be2_ref[...]
    onehot = (batch_ref[...] ==
              lax.broadcasted_iota(jnp.int32, (N_NODES, N_GRAPHS), 1)
              ).astype(jnp.float32)
    sums = lax.dot_general(onehot, h, (((0,), (0,)), ((), ())),
                           preferred_element_type=jnp.float32)
    counts = jnp.sum(onehot, axis=0)
    means = sums / jnp.maximum(counts, 1.0)[:, None]
    out_ref[...] = _dot_t(means, linw_ref[...]) + linb_ref[...]


def _tc(body, out_shapes, *args):
    return pl.pallas_call(body, out_shape=out_shapes)(*args)


def kernel(x, edge_index, batch, W1_rel, b1_rel, W1_root, bn1_gamma, bn1_beta,
           W2_rel, b2_rel, W2_root, W3_rel, b3_rel, W3_root,
           bn2_gamma, bn2_beta, lin_W, lin_b):
    f32 = jnp.float32
    src = edge_index[0].astype(jnp.int32)
    dst = edge_index[1].astype(jnp.int32)
    pad = E_PAD - N_EDGES
    src2d = jnp.concatenate([src, jnp.zeros((pad,), jnp.int32)]).reshape(N_CHUNKS, CH)
    dst2d = jnp.concatenate([dst, jnp.full((pad,), DUMMY_ROW, jnp.int32)]).reshape(N_CHUNKS, CH)
    z64 = jnp.zeros((ACC_ROWS, 64), f32)
    z32 = jnp.zeros((ACC_ROWS, 32), f32)
    batch2 = batch.astype(jnp.int32).reshape(N_NODES, 1)

    w3rel = jnp.pad(W3_rel, ((0, 12), (0, 0)))
    w3root = jnp.pad(W3_root, ((0, 12), (0, 0)))
    b3 = jnp.pad(b3_rel, (0, 12)).reshape(1, 32)
    g2 = jnp.pad(bn2_gamma, (0, 12)).reshape(1, 32)
    be2 = jnp.pad(bn2_beta, (0, 12)).reshape(1, 32)
    linw = jnp.pad(lin_W, ((0, 0), (0, 12)))

    sds = jax.ShapeDtypeStruct

    p1, r1 = _tc(_proj1_body,
                 [sds((N_NODES, 64), f32), sds((N_NODES, 64), f32)],
                 x, W1_rel, W1_root)
    parts1 = _get_agg(64)(p1, src2d, dst2d, z64)
    p2, r2 = _tc(_mid1_body,
                 [sds((N_NODES, 32), f32), sds((N_NODES, 32), f32)],
                 parts1, r1, b1_rel.reshape(1, 64),
                 bn1_gamma.reshape(1, 64), bn1_beta.reshape(1, 64),
                 W2_rel, W2_root)
    parts2 = _get_agg(32)(p2, src2d, dst2d, z32)
    p3, r3 = _tc(_mid2_body,
                 [sds((N_NODES, 32), f32), sds((N_NODES, 32), f32)],
                 parts2, r2, b2_rel.reshape(1, 32), w3rel, w3root)
    parts3 = _get_agg(32)(p3, src2d, dst2d, z32)
    out = _tc(_final_body, sds((N_GRAPHS, 11), f32),
              parts3, r3, b3, g2, be2, batch2,
              linw, lin_b.reshape(1, 11))
    return out


# speedup vs baseline: 7.3588x; 1.0568x over previous
import functools

import jax
import jax.numpy as jnp
from jax import lax
from jax.experimental import pallas as pl
from jax.experimental.pallas import tpu as pltpu
from jax.experimental.pallas import tpu_sc as plsc

N_NODES = 10000
N_EDGES = 320000
N_GRAPHS = 64

CH = 128
CPT = 80
N_TILES = 32
N_CHUNKS = N_TILES * CPT
E_PAD = N_CHUNKS * CH
DUMMY_ROW = N_NODES
ACC_ROWS = 10112
RPT = ACC_ROWS // 16


def _make_agg(d):
    mesh = plsc.VectorSubcoreMesh(core_axis_name="c", subcore_axis_name="s")

    @functools.partial(
        pl.kernel,
        mesh=mesh,
        compiler_params=pltpu.CompilerParams(use_tc_tiling_on_sc=False),
        out_type=jax.ShapeDtypeStruct((2, ACC_ROWS, d), jnp.float32),
        scratch_types=[
            pltpu.VMEM((CPT, CH), jnp.int32),
            pltpu.VMEM((CPT, CH), jnp.int32),
            [pltpu.VMEM((CH, d), jnp.float32) for _ in range(8)],
            pltpu.VMEM_SHARED((ACC_ROWS, d), jnp.float32),
            [pltpu.SemaphoreType.DMA for _ in range(8)],
        ],
    )
    def agg(p_hbm, src_hbm, dst_hbm, z_hbm, out_hbm,
            src_v, dst_v, bufs, acc, sems):
        c = lax.axis_index("c")
        s = lax.axis_index("s")
        wid = c * 16 + s
        pltpu.sync_copy(z_hbm.at[pl.ds(s * RPT, RPT)], acc.at[pl.ds(s * RPT, RPT)])
        pltpu.sync_copy(src_hbm.at[pl.ds(wid * CPT, CPT)], src_v)
        pltpu.sync_copy(dst_hbm.at[pl.ds(wid * CPT, CPT)], dst_v)
        plsc.subcore_barrier()

        def body(jj, carry):
            j = jj * 8
            hs = [pltpu.async_copy(p_hbm.at[src_v.at[j + b]], bufs[b], sems[b])
                  for b in range(8)]
            for b in range(8):
                hs[b].wait()
                pltpu.sync_copy(bufs[b], acc.at[dst_v.at[j + b]], add=True)
            return carry

        lax.fori_loop(0, CPT // 8, body, 0)
        plsc.subcore_barrier()
        pltpu.sync_copy(acc.at[pl.ds(s * RPT, RPT)],
                        out_hbm.at[c, pl.ds(s * RPT, RPT)])

    return agg


_AGG_CACHE = {}


def _get_agg(d):
    if d not in _AGG_CACHE:
        _AGG_CACHE[d] = _make_agg(d)
    return _AGG_CACHE[d]


def _dot_t(a, w):
    return lax.dot_general(a, w, (((1,), (1,)), ((), ())),
                           preferred_element_type=jnp.float32)


def _proj1_body(x_ref, w_rel_ref, w_root_ref, p_ref, r_ref):
    x = x_ref[...]
    p_ref[...] = _dot_t(x, w_rel_ref[...])
    r_ref[...] = _dot_t(x, w_root_ref[...])


def _mid1_body(parts_ref, r1_ref, b1_ref, g1_ref, be1_ref,
               w2rel_ref, w2root_ref, p2_ref, r2_ref):
    agg = parts_ref[0, :N_NODES, :] + parts_ref[1, :N_NODES, :]
    h = agg + b1_ref[...] + r1_ref[...]
    mean = jnp.mean(h, axis=0, keepdims=True)
    var = jnp.mean((h - mean) ** 2, axis=0, keepdims=True)
    h = (h - mean) / jnp.sqrt(var + 1e-5) * g1_ref[...] + be1_ref[...]
    h = jnp.maximum(h, 0.0)
    p2_ref[...] = _dot_t(h, w2rel_ref[...])
    r2_ref[...] = _dot_t(h, w2root_ref[...])


def _mid2_body(parts_ref, r2_ref, b2_ref, w3rel_ref, w3root_ref,
               p3_ref, r3_ref):
    agg = parts_ref[0, :N_NODES, :] + parts_ref[1, :N_NODES, :]
    h = jnp.maximum(agg + b2_ref[...] + r2_ref[...], 0.0)
    p3_ref[...] = _dot_t(h, w3rel_ref[...])
    r3_ref[...] = _dot_t(h, w3root_ref[...])


def _final_body(parts_ref, r3_ref, b3_ref, g2_ref, be2_ref, batch_ref,
                linw_ref, linb_ref, out_ref):
    agg = parts_ref[0, :N_NODES, :] + parts_ref[1, :N_NODES, :]
    h = agg + b3_ref[...] + r3_ref[...]
    mean = jnp.mean(h, axis=0, keepdims=True)
    var = jnp.mean((h - mean) ** 2, axis=0, keepdims=True)
    h = (h - mean) / jnp.sqrt(var + 1e-5) * g2_ref[...] + be2_ref[...]
    onehot = (batch_ref[...] ==
              lax.broadcasted_iota(jnp.int32, (N_NODES, N_GRAPHS), 1)
              ).astype(jnp.float32)
    sums = lax.dot_general(onehot, h, (((0,), (0,)), ((), ())),
                           preferred_element_type=jnp.float32)
    counts = jnp.sum(onehot, axis=0)
    means = sums / jnp.maximum(counts, 1.0)[:, None]
    out_ref[...] = _dot_t(means, linw_ref[...]) + linb_ref[...]


def _tc(body, out_shapes, *args):
    return pl.pallas_call(body, out_shape=out_shapes)(*args)


def kernel(x, edge_index, batch, W1_rel, b1_rel, W1_root, bn1_gamma, bn1_beta,
           W2_rel, b2_rel, W2_root, W3_rel, b3_rel, W3_root,
           bn2_gamma, bn2_beta, lin_W, lin_b):
    f32 = jnp.float32
    src = edge_index[0].astype(jnp.int32)
    dst = edge_index[1].astype(jnp.int32)
    pad = E_PAD - N_EDGES
    src2d = jnp.concatenate([src, jnp.zeros((pad,), jnp.int32)]).reshape(N_CHUNKS, CH)
    dummy = DUMMY_ROW + jnp.arange(pad, dtype=jnp.int32) % (ACC_ROWS - N_NODES)
    dst2d = jnp.concatenate([dst, dummy]).reshape(N_CHUNKS, CH)
    z64 = jnp.zeros((ACC_ROWS, 64), f32)
    z32 = jnp.zeros((ACC_ROWS, 32), f32)
    batch2 = batch.astype(jnp.int32).reshape(N_NODES, 1)

    w3rel = jnp.pad(W3_rel, ((0, 12), (0, 0)))
    w3root = jnp.pad(W3_root, ((0, 12), (0, 0)))
    b3 = jnp.pad(b3_rel, (0, 12)).reshape(1, 32)
    g2 = jnp.pad(bn2_gamma, (0, 12)).reshape(1, 32)
    be2 = jnp.pad(bn2_beta, (0, 12)).reshape(1, 32)
    linw = jnp.pad(lin_W, ((0, 0), (0, 12)))

    sds = jax.ShapeDtypeStruct

    p1, r1 = _tc(_proj1_body,
                 [sds((N_NODES, 64), f32), sds((N_NODES, 64), f32)],
                 x, W1_rel, W1_root)
    parts1 = _get_agg(64)(p1, src2d, dst2d, z64)
    p2, r2 = _tc(_mid1_body,
                 [sds((N_NODES, 32), f32), sds((N_NODES, 32), f32)],
                 parts1, r1, b1_rel.reshape(1, 64),
                 bn1_gamma.reshape(1, 64), bn1_beta.reshape(1, 64),
                 W2_rel, W2_root)
    parts2 = _get_agg(32)(p2, src2d, dst2d, z32)
    p3, r3 = _tc(_mid2_body,
                 [sds((N_NODES, 32), f32), sds((N_NODES, 32), f32)],
                 parts2, r2, b2_rel.reshape(1, 32), w3rel, w3root)
    parts3 = _get_agg(32)(p3, src2d, dst2d, z32)
    out = _tc(_final_body, sds((N_GRAPHS, 11), f32),
              parts3, r3, b3, g2, be2, batch2,
              linw, lin_b.reshape(1, 11))
    return out


# speedup vs baseline: 7.4577x; 1.0134x over previous
import functools

import jax
import jax.numpy as jnp
from jax import lax
from jax.experimental import pallas as pl
from jax.experimental.pallas import tpu as pltpu
from jax.experimental.pallas import tpu_sc as plsc

N_NODES = 10000
N_EDGES = 320000
N_GRAPHS = 64

CH = 128
CPT0 = 120
CPT1 = 40
N_CHUNKS = 16 * (CPT0 + CPT1)
IDX_ROWS = 16 * CPT0 + 15 * CPT1 + CPT0
E_PAD = N_CHUNKS * CH
DUMMY_ROW = N_NODES
ACC_ROWS = 10112
RPT = ACC_ROWS // 16


def _make_agg(d):
    mesh = plsc.VectorSubcoreMesh(core_axis_name="c", subcore_axis_name="s")

    @functools.partial(
        pl.kernel,
        mesh=mesh,
        compiler_params=pltpu.CompilerParams(use_tc_tiling_on_sc=False),
        out_type=jax.ShapeDtypeStruct((2, ACC_ROWS, d), jnp.float32),
        scratch_types=[
            pltpu.VMEM((CPT0, CH), jnp.int32),
            pltpu.VMEM((8, CH), jnp.int32),
            pltpu.VMEM((8, CH), jnp.int32),
            [pltpu.VMEM((CH, d), jnp.float32) for _ in range(8)],
            pltpu.VMEM_SHARED((ACC_ROWS, d), jnp.float32),
            [pltpu.SemaphoreType.DMA for _ in range(8)],
        ],
    )
    def agg(p_hbm, pk_hbm, z_hbm, out_hbm,
            pk_v, src_v, dst_v, bufs, acc, sems):
        c = lax.axis_index("c")
        s = lax.axis_index("s")
        base = (1 - c) * (s * CPT0) + c * (16 * CPT0 + s * CPT1)
        ngroups = (1 - c) * (CPT0 // 8) + c * (CPT1 // 8)
        pltpu.sync_copy(z_hbm.at[pl.ds(s * RPT, RPT)], acc.at[pl.ds(s * RPT, RPT)])
        pltpu.sync_copy(pk_hbm.at[pl.ds(base, CPT0)], pk_v)
        plsc.subcore_barrier()

        def body(jj, carry):
            j = jj * 8
            for b in range(8):
                for k in range(CH // 16):
                    v = pk_v[j + b, pl.ds(k * 16, 16)]
                    src_v[b, pl.ds(k * 16, 16)] = v & 0xFFFF
                    dst_v[b, pl.ds(k * 16, 16)] = lax.shift_right_logical(v, 16)
            hs = [pltpu.async_copy(p_hbm.at[src_v.at[b]], bufs[b], sems[b])
                  for b in range(8)]
            for b in range(8):
                hs[b].wait()
                pltpu.sync_copy(bufs[b], acc.at[dst_v.at[b]], add=True)
            return carry

        lax.fori_loop(0, ngroups, body, 0)
        plsc.subcore_barrier()
        pltpu.sync_copy(acc.at[pl.ds(s * RPT, RPT)],
                        out_hbm.at[c, pl.ds(s * RPT, RPT)])

    return agg


_AGG_CACHE = {}


def _get_agg(d):
    if d not in _AGG_CACHE:
        _AGG_CACHE[d] = _make_agg(d)
    return _AGG_CACHE[d]


def _dot_t(a, w):
    return lax.dot_general(a, w, (((1,), (1,)), ((), ())),
                           preferred_element_type=jnp.float32)


def _proj1_body(x_ref, w_rel_ref, w_root_ref, p_ref, r_ref):
    x = x_ref[...]
    p_ref[...] = _dot_t(x, w_rel_ref[...])
    r_ref[...] = _dot_t(x, w_root_ref[...])


def _mid1_body(parts_ref, r1_ref, b1_ref, g1_ref, be1_ref,
               w2rel_ref, w2root_ref, p2_ref, r2_ref):
    agg = parts_ref[0, :N_NODES, :] + parts_ref[1, :N_NODES, :]
    h = agg + b1_ref[...] + r1_ref[...]
    mean = jnp.mean(h, axis=0, keepdims=True)
    var = jnp.mean((h - mean) ** 2, axis=0, keepdims=True)
    h = (h - mean) / jnp.sqrt(var + 1e-5) * g1_ref[...] + be1_ref[...]
    h = jnp.maximum(h, 0.0)
    p2_ref[...] = _dot_t(h, w2rel_ref[...])
    r2_ref[...] = _dot_t(h, w2root_ref[...])


def _mid2_body(parts_ref, r2_ref, b2_ref, w3rel_ref, w3root_ref,
               p3_ref, r3_ref):
    agg = parts_ref[0, :N_NODES, :] + parts_ref[1, :N_NODES, :]
    h = jnp.maximum(agg + b2_ref[...] + r2_ref[...], 0.0)
    p3_ref[...] = _dot_t(h, w3rel_ref[...])
    r3_ref[...] = _dot_t(h, w3root_ref[...])


def _final_body(parts_ref, r3_ref, b3_ref, g2_ref, be2_ref, batch_ref,
                linw_ref, linb_ref, out_ref):
    agg = parts_ref[0, :N_NODES, :] + parts_ref[1, :N_NODES, :]
    h = agg + b3_ref[...] + r3_ref[...]
    mean = jnp.mean(h, axis=0, keepdims=True)
    var = jnp.mean((h - mean) ** 2, axis=0, keepdims=True)
    h = (h - mean) / jnp.sqrt(var + 1e-5) * g2_ref[...] + be2_ref[...]
    onehot = (batch_ref[...] ==
              lax.broadcasted_iota(jnp.int32, (N_NODES, N_GRAPHS), 1)
              ).astype(jnp.float32)
    sums = lax.dot_general(onehot, h, (((0,), (0,)), ((), ())),
                           preferred_element_type=jnp.float32)
    counts = jnp.sum(onehot, axis=0)
    means = sums / jnp.maximum(counts, 1.0)[:, None]
    out_ref[...] = _dot_t(means, linw_ref[...]) + linb_ref[...]


def _tc(body, out_shapes, *args):
    return pl.pallas_call(body, out_shape=out_shapes)(*args)


def kernel(x, edge_index, batch, W1_rel, b1_rel, W1_root, bn1_gamma, bn1_beta,
           W2_rel, b2_rel, W2_root, W3_rel, b3_rel, W3_root,
           bn2_gamma, bn2_beta, lin_W, lin_b):
    f32 = jnp.float32
    src = edge_index[0].astype(jnp.int32)
    dst = edge_index[1].astype(jnp.int32)
    pad = IDX_ROWS * CH - N_EDGES
    srcp = jnp.concatenate([src, jnp.zeros((pad,), jnp.int32)])
    dummy = DUMMY_ROW + jnp.arange(pad, dtype=jnp.int32) % (ACC_ROWS - N_NODES)
    dstp = jnp.concatenate([dst, dummy])
    packed = (srcp | (dstp << 16)).reshape(IDX_ROWS, CH)
    z64 = jnp.zeros((ACC_ROWS, 64), f32)
    z32 = jnp.zeros((ACC_ROWS, 32), f32)
    batch2 = batch.astype(jnp.int32).reshape(N_NODES, 1)

    w3rel = jnp.pad(W3_rel, ((0, 12), (0, 0)))
    w3root = jnp.pad(W3_root, ((0, 12), (0, 0)))
    b3 = jnp.pad(b3_rel, (0, 12)).reshape(1, 32)
    g2 = jnp.pad(bn2_gamma, (0, 12)).reshape(1, 32)
    be2 = jnp.pad(bn2_beta, (0, 12)).reshape(1, 32)
    linw = jnp.pad(lin_W, ((0, 0), (0, 12)))

    sds = jax.ShapeDtypeStruct

    p1, r1 = _tc(_proj1_body,
                 [sds((N_NODES, 64), f32), sds((N_NODES, 64), f32)],
                 x, W1_rel, W1_root)
    parts1 = _get_agg(64)(p1, packed, z64)
    p2, r2 = _tc(_mid1_body,
                 [sds((N_NODES, 32), f32), sds((N_NODES, 32), f32)],
                 parts1, r1, b1_rel.reshape(1, 64),
                 bn1_gamma.reshape(1, 64), bn1_beta.reshape(1, 64),
                 W2_rel, W2_root)
    parts2 = _get_agg(32)(p2, packed, z32)
    p3, r3 = _tc(_mid2_body,
                 [sds((N_NODES, 32), f32), sds((N_NODES, 32), f32)],
                 parts2, r2, b2_rel.reshape(1, 32), w3rel, w3root)
    parts3 = _get_agg(32)(p3, packed, z32)
    out = _tc(_final_body, sds((N_GRAPHS, 11), f32),
              parts3, r3, b3, g2, be2, batch2,
              linw, lin_b.reshape(1, 11))
    return out
